# baseline (speedup 1.0000x reference)
import functools

import jax
import jax.numpy as jnp
import numpy as np
from jax import lax
from jax.experimental import pallas as pl
from jax.experimental.pallas import tpu as pltpu
from jax.experimental.pallas import tpu_sc as plsc

N = 10000
E = 320000
HID = 128
NUM_RBF = 50
RBF_PAD = 64
L = 4
G = 64
CUTOFF = 4.0
VOCAB = 128

_STEP = np.float32(CUTOFF) * (np.float32(1.0) / np.float32(NUM_RBF - 1))
_COEFF = -0.5 / float(_STEP) ** 2

_W_PERM = np.zeros(HID, dtype=np.int32)
for _j in range(HID // 32):
    for _m in range(16):
        _W_PERM[32 * _j + 2 * _m] = 32 * _j + _m
        _W_PERM[32 * _j + 2 * _m + 1] = 32 * _j + 16 + _m

_NBLK = 1000


def _embed_xl_body(x_ref, emb_ref, lw_ref, lb_ref, h_ref, xl_ref):
    xv = x_ref[0]
    ids = lax.broadcasted_iota(jnp.int32, (VOCAB, _NBLK), 0)
    oh = (ids == jnp.broadcast_to(xv, (VOCAB, _NBLK))).astype(jnp.float32)
    h = lax.dot_general(oh, emb_ref[...], (((0,), (0,)), ((), ())),
                        preferred_element_type=jnp.float32)
    h_ref[...] = h
    xl_ref[...] = jnp.dot(h, lw_ref[...],
                          preferred_element_type=jnp.float32) + lb_ref[...]


def _embed_xl(x3, emb, lw0, lb0):
    return pl.pallas_call(
        _embed_xl_body,
        grid=(N // _NBLK,),
        in_specs=[
            pl.BlockSpec((1, 1, _NBLK), lambda i: (i, 0, 0)),
            pl.BlockSpec((VOCAB, HID), lambda i: (0, 0)),
            pl.BlockSpec((HID, HID), lambda i: (0, 0)),
            pl.BlockSpec((1, HID), lambda i: (0, 0)),
        ],
        out_specs=[
            pl.BlockSpec((_NBLK, HID), lambda i: (i, 0)),
            pl.BlockSpec((_NBLK, HID), lambda i: (i, 0)),
        ],
        out_shape=[
            jax.ShapeDtypeStruct((N, HID), jnp.float32),
            jax.ShapeDtypeStruct((N, HID), jnp.float32),
        ],
    )(x3, emb, lw0, lb0)


_EBLK = 2000


def _filter_body(attr_ref, fw1_ref, fb1_ref, fw2_ref, fb2_ref, o_ref):
    a = attr_ref[0]
    ab = jnp.broadcast_to(a, (RBF_PAD, _EBLK))
    off = lax.broadcasted_iota(jnp.int32, (RBF_PAD, _EBLK), 0).astype(jnp.float32) * _STEP
    d = ab - off
    rbf_t = jnp.exp(_COEFF * d * d)
    t = lax.dot_general(rbf_t, fw1_ref[...], (((0,), (0,)), ((), ())),
                        preferred_element_type=jnp.float32) + fb1_ref[...]
    t = jnp.maximum(t, 0.0)
    w = jnp.dot(t, fw2_ref[...], preferred_element_type=jnp.float32) + fb2_ref[...]
    o_ref[...] = w.astype(jnp.bfloat16)


def _filter(attr3, fw1p, fb1, fw2, fb2):
    return pl.pallas_call(
        _filter_body,
        grid=(E // _EBLK,),
        in_specs=[
            pl.BlockSpec((1, 1, _EBLK), lambda i: (i, 0, 0)),
            pl.BlockSpec((RBF_PAD, HID), lambda i: (0, 0)),
            pl.BlockSpec((1, HID), lambda i: (0, 0)),
            pl.BlockSpec((HID, HID), lambda i: (0, 0)),
            pl.BlockSpec((1, HID), lambda i: (0, 0)),
        ],
        out_specs=pl.BlockSpec((_EBLK, HID), lambda i: (i, 0)),
        out_shape=jax.ShapeDtypeStruct((E, HID), jnp.bfloat16),
    )(attr3, fw1p, fb1, fw2, fb2)


def _ln(h_ref, p0_ref, p1_ref, g_ref, b_ref):
    hn = h_ref[...] + p0_ref[...] + p1_ref[...]
    mu = jnp.mean(hn, axis=-1, keepdims=True)
    cen = hn - mu
    var = jnp.mean(cen * cen, axis=-1, keepdims=True)
    return cen * lax.rsqrt(var + 1e-5) * g_ref[...] + b_ref[...]


def _combine_xl_body(h_ref, p0_ref, p1_ref, g_ref, b_ref, lw_ref, lb_ref,
                     ho_ref, xl_ref):
    hn = _ln(h_ref, p0_ref, p1_ref, g_ref, b_ref)
    ho_ref[...] = hn
    xl_ref[...] = jnp.dot(hn, lw_ref[...],
                          preferred_element_type=jnp.float32) + lb_ref[...]


def _combine_xl(h, p0, p1, g, b, lw, lb):
    return pl.pallas_call(
        _combine_xl_body,
        grid=(N // _NBLK,),
        in_specs=[
            pl.BlockSpec((_NBLK, HID), lambda i: (i, 0)),
            pl.BlockSpec((_NBLK, HID), lambda i: (i, 0)),
            pl.BlockSpec((_NBLK, HID), lambda i: (i, 0)),
            pl.BlockSpec((1, HID), lambda i: (0, 0)),
            pl.BlockSpec((1, HID), lambda i: (0, 0)),
            pl.BlockSpec((HID, HID), lambda i: (0, 0)),
            pl.BlockSpec((1, HID), lambda i: (0, 0)),
        ],
        out_specs=[
            pl.BlockSpec((_NBLK, HID), lambda i: (i, 0)),
            pl.BlockSpec((_NBLK, HID), lambda i: (i, 0)),
        ],
        out_shape=[
            jax.ShapeDtypeStruct((N, HID), jnp.float32),
            jax.ShapeDtypeStruct((N, HID), jnp.float32),
        ],
    )(h, p0, p1, g, b, lw, lb)


def _combine_pool_body(h_ref, p0_ref, p1_ref, g_ref, b_ref, b3_ref, o_ref,
                       acc, cnt):
    i = pl.program_id(0)

    @pl.when(i == 0)
    def _():
        acc[...] = jnp.zeros((G, HID), jnp.float32)
        cnt[...] = jnp.zeros((G, HID), jnp.float32)

    hn = _ln(h_ref, p0_ref, p1_ref, g_ref, b_ref)
    bv = b3_ref[0]
    gids = lax.broadcasted_iota(jnp.int32, (G, _NBLK), 0)
    oh = (gids == jnp.broadcast_to(bv, (G, _NBLK))).astype(jnp.float32)
    acc[...] += jnp.dot(oh, hn, preferred_element_type=jnp.float32)
    cnt[...] += jnp.broadcast_to(jnp.sum(oh, axis=1, keepdims=True), (G, HID))

    @pl.when(i == pl.num_programs(0) - 1)
    def _():
        o_ref[...] = acc[...] / jnp.maximum(cnt[...], 1.0)


def _combine_pool(h, p0, p1, g, b, b3):
    return pl.pallas_call(
        _combine_pool_body,
        grid=(N // _NBLK,),
        in_specs=[
            pl.BlockSpec((_NBLK, HID), lambda i: (i, 0)),
            pl.BlockSpec((_NBLK, HID), lambda i: (i, 0)),
            pl.BlockSpec((_NBLK, HID), lambda i: (i, 0)),
            pl.BlockSpec((1, HID), lambda i: (0, 0)),
            pl.BlockSpec((1, HID), lambda i: (0, 0)),
            pl.BlockSpec((1, 1, _NBLK), lambda i: (i, 0, 0)),
        ],
        out_specs=pl.BlockSpec((G, HID), lambda i: (0, 0)),
        out_shape=jax.ShapeDtypeStruct((G, HID), jnp.float32),
        scratch_shapes=[
            pltpu.VMEM((G, HID), jnp.float32),
            pltpu.VMEM((G, HID), jnp.float32),
        ],
    )(h, p0, p1, g, b, b3)


_NCORE = 2
_NSUB = 16
_EC = 80
_E_PER_TILE = E // (_NCORE * _NSUB)
_NCHUNK = _E_PER_TILE // _EC
_NPAD = 10240
_R_PER_TILE = _NPAD // _NSUB
_ZR = 128


_NBUF = 2


def _sc_body(xl_hbm, w_hbm, col_hbm, row_hbm, out_hbm, acc, *scr):
    col_v = scr[0:_NBUF]
    row_v = scr[_NBUF:2 * _NBUF]
    row_s = scr[2 * _NBUF:3 * _NBUF]
    w_v = scr[3 * _NBUF:4 * _NBUF]
    g_v = scr[4 * _NBUF:5 * _NBUF]
    sem_i = scr[5 * _NBUF:6 * _NBUF]
    sem_w = scr[6 * _NBUF:7 * _NBUF]
    sem_g = scr[7 * _NBUF:8 * _NBUF]
    sem_s = scr[8 * _NBUF:9 * _NBUF]

    cid = lax.axis_index("c")
    sid = lax.axis_index("s")

    def _zrow(i, carry):
        for j in range(HID // 16):
            g_v[0][i, pl.ds(j * 16, 16)] = jnp.zeros((16,), jnp.float32)
        return carry
    lax.fori_loop(0, _EC, _zrow, 0)
    for t in range(_EC // 16):
        row_s[1][pl.ds(t * 16, 16)] = jnp.zeros((16,), jnp.int32)
    for t in range(_R_PER_TILE // _EC):
        pltpu.sync_copy(g_v[0], acc.at[pl.ds(sid * _R_PER_TILE + t * _EC, _EC)])
    pltpu.async_copy(g_v[0], acc.at[row_s[1]], sem_s[1], add=True)
    plsc.subcore_barrier()

    base = cid * (E // _NCORE) + sid * _E_PER_TILE

    def _issue_loads(c, b):
        off = base + c * _EC
        pltpu.async_copy(col_hbm.at[pl.ds(off, _EC)], col_v[b], sem_i[b])
        pltpu.async_copy(row_hbm.at[pl.ds(off, _EC)], row_v[b], sem_i[b])
        pltpu.async_copy(w_hbm.at[pl.ds(off, _EC)], w_v[b], sem_w[b])

    def _wait_idx(b):
        pltpu.make_async_copy(col_hbm.at[pl.ds(0, _EC)], col_v[b], sem_i[b]).wait()
        pltpu.make_async_copy(row_hbm.at[pl.ds(0, _EC)], row_v[b], sem_i[b]).wait()

    def _wait_w(b):
        pltpu.make_async_copy(w_hbm.at[pl.ds(0, _EC)], w_v[b], sem_w[b]).wait()

    def _issue_gather(b):
        pltpu.async_copy(xl_hbm.at[col_v[b]], g_v[b], sem_g[b])

    def _wait_gather(b):
        pltpu.make_async_copy(xl_hbm.at[col_v[b]], g_v[b], sem_g[b]).wait()

    def _issue_scatter(b):
        pltpu.async_copy(g_v[b], acc.at[row_s[b]], sem_s[b], add=True)

    def _wait_scatter(b):
        pltpu.make_async_copy(g_v[b], acc.at[row_s[b]], sem_s[b]).wait()

    def _mul(b):
        def _m(q, c2):
            for u in range(4):
                e = q * 4 + u
                for j in range(HID // 32):
                    wi = w_v[b][e, pl.ds(j * 16, 16)]
                    wa = lax.shift_left(wi, 16).view(jnp.float32)
                    wb = lax.bitwise_and(wi, jnp.int32(-65536)).view(jnp.float32)
                    sa = pl.ds(j * 32, 16)
                    sb = pl.ds(j * 32 + 16, 16)
                    g_v[b][e, sa] = g_v[b][e, sa] * wa
                    g_v[b][e, sb] = g_v[b][e, sb] * wb
            return c2
        lax.fori_loop(0, _EC // 4, _m, 0)

    def _copy_row(b):
        for t in range(_EC // 16):
            s = pl.ds(t * 16, 16)
            row_s[b][s] = row_v[b][s]

    def _process(b):
        _wait_gather(b)
        _wait_w(b)
        _mul(b)
        _copy_row(b)
        _issue_scatter(b)

    _issue_loads(0, 0)
    _issue_loads(1, 1)
    _wait_idx(0)
    _issue_gather(0)

    def _iter(k, carry):
        c2 = lax.rem(2 * k + 2, _NCHUNK)
        c3 = lax.rem(2 * k + 3, _NCHUNK)
        _wait_idx(1)
        _wait_scatter(1)
        _issue_gather(1)
        _process(0)
        _issue_loads(c2, 0)
        _process(1)
        _issue_loads(c3, 1)
        _wait_idx(0)
        _wait_scatter(0)
        _issue_gather(0)
        return carry

    lax.fori_loop(0, _NCHUNK // 2, _iter, 0)
    _process(0)
    _wait_scatter(0)
    _wait_scatter(1)
    _wait_idx(1)
    _wait_w(1)
    plsc.subcore_barrier()

    for t in range(_R_PER_TILE // _ZR):
        r0 = sid * _R_PER_TILE + t * _ZR
        pltpu.sync_copy(acc.at[pl.ds(r0, _ZR)], out_hbm.at[cid, pl.ds(r0, _ZR)])


@functools.partial(jax.jit, static_argnames=())
def _sc_scatter(xl, w, col, row):
    mesh = plsc.VectorSubcoreMesh(core_axis_name="c", subcore_axis_name="s")
    kfn = pl.kernel(
        _sc_body,
        mesh=mesh,
        out_type=jax.ShapeDtypeStruct((_NCORE, _NPAD, HID), jnp.float32),
        scratch_types=(
            [pltpu.VMEM_SHARED((_NPAD, HID), jnp.float32)]
            + [pltpu.VMEM((_EC,), jnp.int32) for _ in range(3 * _NBUF)]
            + [pltpu.VMEM((_EC, HID // 2), jnp.int32) for _ in range(_NBUF)]
            + [pltpu.VMEM((_EC, HID), jnp.float32) for _ in range(_NBUF)]
            + [pltpu.SemaphoreType.DMA for _ in range(4 * _NBUF)]
        ),
    )
    return kfn(xl, w, col, row)


def kernel(x, edge_index, edge_attr, batch, emb, fw1, fb1, fw2, fb2, lw, lb, ln_g, ln_b):
    x3 = x.astype(jnp.int32).reshape(N // _NBLK, 1, _NBLK)
    b3 = batch.astype(jnp.int32).reshape(N // _NBLK, 1, _NBLK)
    attr3 = edge_attr.reshape(E // _EBLK, 1, _EBLK)
    row = edge_index[0].astype(jnp.int32)
    col = edge_index[1].astype(jnp.int32)
    fw1p = jnp.pad(fw1, ((0, 0), (0, RBF_PAD - NUM_RBF), (0, 0)))
    fw2p = fw2[:, :, _W_PERM]
    fb2p = fb2[:, _W_PERM]

    ws = [_filter(attr3, fw1p[i], fb1[i][None, :], fw2p[i], fb2p[i][None, :])
          for i in range(L)]
    h, xl = _embed_xl(x3, emb, lw[0], lb[0][None, :])
    wpk = [lax.bitcast_convert_type(w.reshape(E, HID // 2, 2), jnp.int32)
           for w in ws]
    for i in range(L):
        parts = _sc_scatter(xl, wpk[i], col, row)
        if i + 1 < L:
            h, xl = _combine_xl(h, parts[0, :N], parts[1, :N],
                                ln_g[i][None, :], ln_b[i][None, :],
                                lw[i + 1], lb[i + 1][None, :])
        else:
            return _combine_pool(h, parts[0, :N], parts[1, :N],
                                 ln_g[i][None, :], ln_b[i][None, :], b3)


# speedup vs baseline: 4.0677x; 4.0677x over previous
import functools

import jax
import jax.numpy as jnp
import numpy as np
from jax import lax
from jax.experimental import pallas as pl
from jax.experimental.pallas import tpu as pltpu
from jax.experimental.pallas import tpu_sc as plsc

N = 10000
E = 320000
HID = 128
NUM_RBF = 50
RBF_PAD = 64
L = 4
G = 64
CUTOFF = 4.0
VOCAB = 128

_STEP = np.float32(CUTOFF) * (np.float32(1.0) / np.float32(NUM_RBF - 1))
_COEFF = -0.5 / float(_STEP) ** 2

_PA = np.concatenate([np.arange(32 * j, 32 * j + 16) for j in range(HID // 32)])
_PB = _PA + 16

_NBLK = 1000


def _embed_xl_body(x_ref, emb_ref, lw_ref, lb_ref, h_ref, xl_ref):
    xv = x_ref[0]
    ids = lax.broadcasted_iota(jnp.int32, (VOCAB, _NBLK), 0)
    oh = (ids == jnp.broadcast_to(xv, (VOCAB, _NBLK))).astype(jnp.float32)
    h = lax.dot_general(oh, emb_ref[...], (((0,), (0,)), ((), ())),
                        preferred_element_type=jnp.float32)
    h_ref[...] = h
    xl_ref[...] = jnp.dot(h, lw_ref[...],
                          preferred_element_type=jnp.float32) + lb_ref[...]


def _embed_xl(x3, emb, lw0, lb0):
    return pl.pallas_call(
        _embed_xl_body,
        grid=(N // _NBLK,),
        in_specs=[
            pl.BlockSpec((1, 1, _NBLK), lambda i: (i, 0, 0)),
            pl.BlockSpec((VOCAB, HID), lambda i: (0, 0)),
            pl.BlockSpec((HID, HID), lambda i: (0, 0)),
            pl.BlockSpec((1, HID), lambda i: (0, 0)),
        ],
        out_specs=[
            pl.BlockSpec((_NBLK, HID), lambda i: (i, 0)),
            pl.BlockSpec((_NBLK, HID), lambda i: (i, 0)),
        ],
        out_shape=[
            jax.ShapeDtypeStruct((N, HID), jnp.float32),
            jax.ShapeDtypeStruct((N, HID), jnp.float32),
        ],
    )(x3, emb, lw0, lb0)


_EBLK = 2000


def _filter_body(attr_ref, fw1_ref, fb1_ref, fw2a_ref, fb2a_ref, fw2b_ref,
                 fb2b_ref, o_ref):
    a = attr_ref[0]
    ab = jnp.broadcast_to(a, (RBF_PAD, _EBLK))
    off = lax.broadcasted_iota(jnp.int32, (RBF_PAD, _EBLK), 0).astype(jnp.float32) * _STEP
    d = ab - off
    rbf_t = jnp.exp(_COEFF * d * d)
    t = lax.dot_general(rbf_t, fw1_ref[...], (((0,), (0,)), ((), ())),
                        preferred_element_type=jnp.float32) + fb1_ref[...]
    t = jnp.maximum(t, 0.0)
    wa = jnp.dot(t, fw2a_ref[...], preferred_element_type=jnp.float32) + fb2a_ref[...]
    wb = jnp.dot(t, fw2b_ref[...], preferred_element_type=jnp.float32) + fb2b_ref[...]
    ia = lax.bitcast_convert_type(wa, jnp.int32) + jnp.int32(0x8000)
    ib = lax.bitcast_convert_type(wb, jnp.int32) + jnp.int32(0x8000)
    lo = lax.shift_right_logical(ia, 16)
    hi = lax.bitwise_and(ib, jnp.int32(-65536))
    o_ref[...] = lax.bitwise_or(lo, hi)


def _filter(attr3, fw1p, fb1, fw2a, fb2a, fw2b, fb2b):
    return pl.pallas_call(
        _filter_body,
        grid=(E // _EBLK,),
        in_specs=[
            pl.BlockSpec((1, 1, _EBLK), lambda i: (i, 0, 0)),
            pl.BlockSpec((RBF_PAD, HID), lambda i: (0, 0)),
            pl.BlockSpec((1, HID), lambda i: (0, 0)),
            pl.BlockSpec((HID, HID // 2), lambda i: (0, 0)),
            pl.BlockSpec((1, HID // 2), lambda i: (0, 0)),
            pl.BlockSpec((HID, HID // 2), lambda i: (0, 0)),
            pl.BlockSpec((1, HID // 2), lambda i: (0, 0)),
        ],
        out_specs=pl.BlockSpec((_EBLK, HID // 2), lambda i: (i, 0)),
        out_shape=jax.ShapeDtypeStruct((E, HID // 2), jnp.int32),
    )(attr3, fw1p, fb1, fw2a, fb2a, fw2b, fb2b)


def _ln(h_ref, p0_ref, p1_ref, g_ref, b_ref):
    hn = h_ref[...] + p0_ref[...] + p1_ref[...]
    mu = jnp.mean(hn, axis=-1, keepdims=True)
    cen = hn - mu
    var = jnp.mean(cen * cen, axis=-1, keepdims=True)
    return cen * lax.rsqrt(var + 1e-5) * g_ref[...] + b_ref[...]


def _combine_xl_body(h_ref, p0_ref, p1_ref, g_ref, b_ref, lw_ref, lb_ref,
                     ho_ref, xl_ref):
    hn = _ln(h_ref, p0_ref, p1_ref, g_ref, b_ref)
    ho_ref[...] = hn
    xl_ref[...] = jnp.dot(hn, lw_ref[...],
                          preferred_element_type=jnp.float32) + lb_ref[...]


def _combine_xl(h, p0, p1, g, b, lw, lb):
    return pl.pallas_call(
        _combine_xl_body,
        grid=(N // _NBLK,),
        in_specs=[
            pl.BlockSpec((_NBLK, HID), lambda i: (i, 0)),
            pl.BlockSpec((_NBLK, HID), lambda i: (i, 0)),
            pl.BlockSpec((_NBLK, HID), lambda i: (i, 0)),
            pl.BlockSpec((1, HID), lambda i: (0, 0)),
            pl.BlockSpec((1, HID), lambda i: (0, 0)),
            pl.BlockSpec((HID, HID), lambda i: (0, 0)),
            pl.BlockSpec((1, HID), lambda i: (0, 0)),
        ],
        out_specs=[
            pl.BlockSpec((_NBLK, HID), lambda i: (i, 0)),
            pl.BlockSpec((_NBLK, HID), lambda i: (i, 0)),
        ],
        out_shape=[
            jax.ShapeDtypeStruct((N, HID), jnp.float32),
            jax.ShapeDtypeStruct((N, HID), jnp.float32),
        ],
    )(h, p0, p1, g, b, lw, lb)


def _combine_pool_body(h_ref, p0_ref, p1_ref, g_ref, b_ref, b3_ref, o_ref,
                       acc, cnt):
    i = pl.program_id(0)

    @pl.when(i == 0)
    def _():
        acc[...] = jnp.zeros((G, HID), jnp.float32)
        cnt[...] = jnp.zeros((G, HID), jnp.float32)

    hn = _ln(h_ref, p0_ref, p1_ref, g_ref, b_ref)
    bv = b3_ref[0]
    gids = lax.broadcasted_iota(jnp.int32, (G, _NBLK), 0)
    oh = (gids == jnp.broadcast_to(bv, (G, _NBLK))).astype(jnp.float32)
    acc[...] += jnp.dot(oh, hn, preferred_element_type=jnp.float32)
    cnt[...] += jnp.broadcast_to(jnp.sum(oh, axis=1, keepdims=True), (G, HID))

    @pl.when(i == pl.num_programs(0) - 1)
    def _():
        o_ref[...] = acc[...] / jnp.maximum(cnt[...], 1.0)


def _combine_pool(h, p0, p1, g, b, b3):
    return pl.pallas_call(
        _combine_pool_body,
        grid=(N // _NBLK,),
        in_specs=[
            pl.BlockSpec((_NBLK, HID), lambda i: (i, 0)),
            pl.BlockSpec((_NBLK, HID), lambda i: (i, 0)),
            pl.BlockSpec((_NBLK, HID), lambda i: (i, 0)),
            pl.BlockSpec((1, HID), lambda i: (0, 0)),
            pl.BlockSpec((1, HID), lambda i: (0, 0)),
            pl.BlockSpec((1, 1, _NBLK), lambda i: (i, 0, 0)),
        ],
        out_specs=pl.BlockSpec((G, HID), lambda i: (0, 0)),
        out_shape=jax.ShapeDtypeStruct((G, HID), jnp.float32),
        scratch_shapes=[
            pltpu.VMEM((G, HID), jnp.float32),
            pltpu.VMEM((G, HID), jnp.float32),
        ],
    )(h, p0, p1, g, b, b3)


_NCORE = 2
_NSUB = 16
_EC = 80
_E_PER_TILE = E // (_NCORE * _NSUB)
_NCHUNK = _E_PER_TILE // _EC
_NPAD = 10240
_R_PER_TILE = _NPAD // _NSUB
_ZR = 128


_NBUF = 2


def _sc_body(xl_hbm, w_hbm, col_hbm, row_hbm, out_hbm, acc, *scr):
    col_v = scr[0:_NBUF]
    row_v = scr[_NBUF:2 * _NBUF]
    row_s = scr[2 * _NBUF:3 * _NBUF]
    w_v = scr[3 * _NBUF:4 * _NBUF]
    g_v = scr[4 * _NBUF:5 * _NBUF]
    sem_i = scr[5 * _NBUF:6 * _NBUF]
    sem_w = scr[6 * _NBUF:7 * _NBUF]
    sem_g = scr[7 * _NBUF:8 * _NBUF]
    sem_s = scr[8 * _NBUF:9 * _NBUF]

    cid = lax.axis_index("c")
    sid = lax.axis_index("s")

    def _zrow(i, carry):
        for j in range(HID // 16):
            g_v[0][i, pl.ds(j * 16, 16)] = jnp.zeros((16,), jnp.float32)
        return carry
    lax.fori_loop(0, _EC, _zrow, 0)
    for t in range(_EC // 16):
        row_s[1][pl.ds(t * 16, 16)] = jnp.zeros((16,), jnp.int32)
    for t in range(_R_PER_TILE // _EC):
        pltpu.sync_copy(g_v[0], acc.at[pl.ds(sid * _R_PER_TILE + t * _EC, _EC)])
    pltpu.async_copy(g_v[0], acc.at[row_s[1]], sem_s[1], add=True)
    plsc.subcore_barrier()

    base = cid * (E // _NCORE) + sid * _E_PER_TILE

    def _issue_loads(c, b):
        off = base + c * _EC
        pltpu.async_copy(col_hbm.at[pl.ds(off, _EC)], col_v[b], sem_i[b])
        pltpu.async_copy(row_hbm.at[pl.ds(off, _EC)], row_v[b], sem_i[b])
        pltpu.async_copy(w_hbm.at[pl.ds(off, _EC)], w_v[b], sem_w[b])

    def _wait_idx(b):
        pltpu.make_async_copy(col_hbm.at[pl.ds(0, _EC)], col_v[b], sem_i[b]).wait()
        pltpu.make_async_copy(row_hbm.at[pl.ds(0, _EC)], row_v[b], sem_i[b]).wait()

    def _wait_w(b):
        pltpu.make_async_copy(w_hbm.at[pl.ds(0, _EC)], w_v[b], sem_w[b]).wait()

    def _issue_gather(b):
        pltpu.async_copy(xl_hbm.at[col_v[b]], g_v[b], sem_g[b])

    def _wait_gather(b):
        pltpu.make_async_copy(xl_hbm.at[col_v[b]], g_v[b], sem_g[b]).wait()

    def _issue_scatter(b):
        pltpu.async_copy(g_v[b], acc.at[row_s[b]], sem_s[b], add=True)

    def _wait_scatter(b):
        pltpu.make_async_copy(g_v[b], acc.at[row_s[b]], sem_s[b]).wait()

    def _mul(b):
        def _m(q, c2):
            for u in range(4):
                e = q * 4 + u
                for j in range(HID // 32):
                    wi = w_v[b][e, pl.ds(j * 16, 16)]
                    wa = lax.shift_left(wi, 16).view(jnp.float32)
                    wb = lax.bitwise_and(wi, jnp.int32(-65536)).view(jnp.float32)
                    sa = pl.ds(j * 32, 16)
                    sb = pl.ds(j * 32 + 16, 16)
                    g_v[b][e, sa] = g_v[b][e, sa] * wa
                    g_v[b][e, sb] = g_v[b][e, sb] * wb
            return c2
        lax.fori_loop(0, _EC // 4, _m, 0)

    def _copy_row(b):
        for t in range(_EC // 16):
            s = pl.ds(t * 16, 16)
            row_s[b][s] = row_v[b][s]

    def _process(b):
        _wait_gather(b)
        _wait_w(b)
        _mul(b)
        _copy_row(b)
        _issue_scatter(b)

    _issue_loads(0, 0)
    _issue_loads(1, 1)
    _wait_idx(0)
    _issue_gather(0)

    def _iter(k, carry):
        c2 = lax.rem(2 * k + 2, _NCHUNK)
        c3 = lax.rem(2 * k + 3, _NCHUNK)
        _wait_idx(1)
        _wait_scatter(1)
        _issue_gather(1)
        _process(0)
        _issue_loads(c2, 0)
        _process(1)
        _issue_loads(c3, 1)
        _wait_idx(0)
        _wait_scatter(0)
        _issue_gather(0)
        return carry

    lax.fori_loop(0, _NCHUNK // 2, _iter, 0)
    _process(0)
    _wait_scatter(0)
    _wait_scatter(1)
    _wait_idx(1)
    _wait_w(1)
    plsc.subcore_barrier()

    for t in range(_R_PER_TILE // _ZR):
        r0 = sid * _R_PER_TILE + t * _ZR
        pltpu.sync_copy(acc.at[pl.ds(r0, _ZR)], out_hbm.at[cid, pl.ds(r0, _ZR)])


@functools.partial(jax.jit, static_argnames=())
def _sc_scatter(xl, w, col, row):
    mesh = plsc.VectorSubcoreMesh(core_axis_name="c", subcore_axis_name="s")
    kfn = pl.kernel(
        _sc_body,
        mesh=mesh,
        out_type=jax.ShapeDtypeStruct((_NCORE, _NPAD, HID), jnp.float32),
        scratch_types=(
            [pltpu.VMEM_SHARED((_NPAD, HID), jnp.float32)]
            + [pltpu.VMEM((_EC,), jnp.int32) for _ in range(3 * _NBUF)]
            + [pltpu.VMEM((_EC, HID // 2), jnp.int32) for _ in range(_NBUF)]
            + [pltpu.VMEM((_EC, HID), jnp.float32) for _ in range(_NBUF)]
            + [pltpu.SemaphoreType.DMA for _ in range(4 * _NBUF)]
        ),
    )
    return kfn(xl, w, col, row)


def kernel(x, edge_index, edge_attr, batch, emb, fw1, fb1, fw2, fb2, lw, lb, ln_g, ln_b):
    x3 = x.astype(jnp.int32).reshape(N // _NBLK, 1, _NBLK)
    b3 = batch.astype(jnp.int32).reshape(N // _NBLK, 1, _NBLK)
    attr3 = edge_attr.reshape(E // _EBLK, 1, _EBLK)
    row = edge_index[0].astype(jnp.int32)
    col = edge_index[1].astype(jnp.int32)
    fw1p = jnp.pad(fw1, ((0, 0), (0, RBF_PAD - NUM_RBF), (0, 0)))
    fw2a, fb2a = fw2[:, :, _PA], fb2[:, _PA]
    fw2b, fb2b = fw2[:, :, _PB], fb2[:, _PB]

    ws = [_filter(attr3, fw1p[i], fb1[i][None, :], fw2a[i], fb2a[i][None, :],
                  fw2b[i], fb2b[i][None, :])
          for i in range(L)]
    h, xl = _embed_xl(x3, emb, lw[0], lb[0][None, :])
    for i in range(L):
        parts = _sc_scatter(xl, ws[i], col, row)
        if i + 1 < L:
            h, xl = _combine_xl(h, parts[0, :N], parts[1, :N],
                                ln_g[i][None, :], ln_b[i][None, :],
                                lw[i + 1], lb[i + 1][None, :])
        else:
            return _combine_pool(h, parts[0, :N], parts[1, :N],
                                 ln_g[i][None, :], ln_b[i][None, :], b3)
